# baseline (device time: 23387 ns/iter reference)
import jax
import jax.numpy as jnp
from jax import lax
from jax.experimental import pallas as pl
from jax.experimental.pallas import tpu as pltpu

N_DEV = 8
B, SQ, SKV = 2, 256, 256
HQ_LOCAL, DH = 4, 64
D_MODEL = 512
RQ = SQ // N_DEV


def kernel(x, Wq, K_ext, V_ext, Wo):
    my = lax.axis_index("i")

    def body(my_ref, x_ref, wq_ref, k_ref, v_ref, wo_ref, out_ref,
             send_buf, recv_buf, red_buf, s1, r1, s2, r2):
        my_pos = lax.axis_index("i")

        barrier_sem = pltpu.get_barrier_semaphore()
        for o in range(1, N_DEV):
            pl.semaphore_signal(
                barrier_sem, inc=1,
                device_id=(lax.rem(my_pos + o, N_DEV),),
                device_id_type=pl.DeviceIdType.MESH,
            )
        pl.semaphore_wait(barrier_sem, N_DEV - 1)

        wq = (wq_ref[...] * 0.125).astype(jnp.bfloat16)
        wo = wo_ref[...].astype(jnp.bfloat16)
        qi = lax.broadcasted_iota(jnp.int32, (SQ, SKV), 0)
        ki = lax.broadcasted_iota(jnp.int32, (SQ, SKV), 1)
        mask = (jnp.abs(qi - ki) <= 128) | (ki < 32) | (qi < 32)

        started = []

        def send(src, dst, send_sem, recv_sem, tgt):
            rdma = pltpu.make_async_remote_copy(
                src_ref=src, dst_ref=dst, send_sem=send_sem,
                recv_sem=recv_sem, device_id=(tgt,),
                device_id_type=pl.DeviceIdType.MESH,
            )
            rdma.start()
            started.append(rdma)

        def wait_recv(dst, recv_sem):
            pltpu.make_async_remote_copy(
                src_ref=dst, dst_ref=dst, send_sem=recv_sem,
                recv_sem=recv_sem, device_id=(my_pos,),
                device_id_type=pl.DeviceIdType.MESH,
            ).wait_recv()

        for b in range(B):
            xb = x_ref[b].astype(jnp.bfloat16)
            q = jnp.dot(xb, wq, preferred_element_type=jnp.float32)
            ctx_parts = []
            for h in range(HQ_LOCAL):
                qh = q[:, h * DH:(h + 1) * DH].astype(jnp.bfloat16)
                kh = k_ref[b, :, h * DH:(h + 1) * DH].astype(
                    jnp.bfloat16)
                s = lax.dot_general(
                    qh, kh, (((1,), (1,)), ((), ())),
                    preferred_element_type=jnp.float32,
                )
                w = jnp.exp(jnp.where(mask, s, -1e9))
                w = w / jnp.sum(w, axis=-1, keepdims=True)
                vh = v_ref[b, :, h * DH:(h + 1) * DH].astype(jnp.bfloat16)
                ctx_parts.append(jnp.dot(
                    w.astype(jnp.bfloat16), vh,
                    preferred_element_type=jnp.float32,
                ))
            ctx = jnp.concatenate(ctx_parts, axis=-1).astype(jnp.bfloat16)
            part = jnp.dot(ctx, wo, preferred_element_type=jnp.float32)
            for t in range(N_DEV):
                send_buf[t, b] = part[t * RQ:(t + 1) * RQ, :].astype(
                    jnp.bfloat16)
            for o in range(1, N_DEV):
                tgt = lax.rem(my_pos + o, N_DEV)
                send(send_buf.at[tgt, b], recv_buf.at[my_pos, b],
                     s1.at[o, b], r1.at[my_pos, b], tgt)

        for b in range(B):
            acc = send_buf[my_pos, b].astype(jnp.float32)
            for o in range(1, N_DEV):
                src = lax.rem(my_pos + N_DEV - o, N_DEV)
                wait_recv(recv_buf.at[src, b], r1.at[src, b])
                acc = acc + recv_buf[src, b].astype(jnp.float32)
            red_buf[b] = acc.astype(jnp.bfloat16)
            out_ref[b, pl.ds(my_pos * RQ, RQ), :] = red_buf[b]
            for o in range(1, N_DEV):
                tgt = lax.rem(my_pos + o, N_DEV)
                send(red_buf.at[b], out_ref.at[b, pl.ds(my_pos * RQ, RQ), :],
                     s2.at[o, b], r2.at[my_pos, b], tgt)

        for b in range(B):
            for o in range(1, N_DEV):
                src = lax.rem(my_pos + N_DEV - o, N_DEV)
                wait_recv(out_ref.at[b, pl.ds(src * RQ, RQ), :], r2.at[src, b])

        for rdma in started:
            rdma.wait_send()

    grid_spec = pltpu.PrefetchScalarGridSpec(
        num_scalar_prefetch=1,
        grid=(1,),
        in_specs=[
            pl.BlockSpec((B, SQ, D_MODEL), lambda i, s: (0, 0, 0)),
            pl.BlockSpec((D_MODEL, HQ_LOCAL * DH), lambda i, s: (0, 0)),
            pl.BlockSpec((B, SKV, HQ_LOCAL * DH),
                         lambda i, s: (0, 0, s[0])),
            pl.BlockSpec((B, SKV, HQ_LOCAL * DH),
                         lambda i, s: (0, 0, s[0])),
            pl.BlockSpec((HQ_LOCAL * DH, D_MODEL), lambda i, s: (0, 0)),
        ],
        out_specs=pl.BlockSpec((B, SQ, D_MODEL), lambda i, s: (0, 0, 0)),
        scratch_shapes=[
            pltpu.VMEM((N_DEV, B, RQ, D_MODEL), jnp.bfloat16),
            pltpu.VMEM((N_DEV, B, RQ, D_MODEL), jnp.bfloat16),
            pltpu.VMEM((B, RQ, D_MODEL), jnp.bfloat16),
            pltpu.SemaphoreType.DMA((N_DEV, B)),
            pltpu.SemaphoreType.DMA((N_DEV, B)),
            pltpu.SemaphoreType.DMA((N_DEV, B)),
            pltpu.SemaphoreType.DMA((N_DEV, B)),
        ],
    )

    return pl.pallas_call(
        body,
        grid_spec=grid_spec,
        out_shape=jax.ShapeDtypeStruct((B, SQ, D_MODEL), jnp.bfloat16),
        compiler_params=pltpu.CompilerParams(collective_id=0),
    )(jnp.reshape(my, (1,)).astype(jnp.int32), x, Wq,
      jnp.reshape(K_ext, (B, SKV, 32 * DH)),
      jnp.reshape(V_ext, (B, SKV, 32 * DH)), Wo)


# device time: 21867 ns/iter; 1.0695x vs baseline; 1.0695x over previous
import jax
import jax.numpy as jnp
from jax import lax
from jax.experimental import pallas as pl
from jax.experimental.pallas import tpu as pltpu

N_DEV = 8
B, SQ, SKV = 2, 256, 256
HQ_LOCAL, DH = 4, 64
D_MODEL = 512
RQ = SQ // N_DEV


def kernel(x, Wq, K_ext, V_ext, Wo):
    my = lax.axis_index("i")
    h0 = my * HQ_LOCAL
    K_loc = jnp.transpose(
        lax.dynamic_slice_in_dim(K_ext, h0, HQ_LOCAL, axis=2), (0, 2, 1, 3)
    ).astype(jnp.bfloat16)
    V_loc = jnp.transpose(
        lax.dynamic_slice_in_dim(V_ext, h0, HQ_LOCAL, axis=2), (0, 2, 1, 3)
    ).astype(jnp.bfloat16)

    def body(x_ref, wq_ref, k_ref, v_ref, wo_ref, out_ref,
             send_buf, recv_buf, red_buf, s1, r1, s2, r2):
        my_pos = lax.axis_index("i")

        barrier_sem = pltpu.get_barrier_semaphore()
        for o in range(1, N_DEV):
            pl.semaphore_signal(
                barrier_sem, inc=1,
                device_id=(lax.rem(my_pos + o, N_DEV),),
                device_id_type=pl.DeviceIdType.MESH,
            )
        pl.semaphore_wait(barrier_sem, N_DEV - 1)

        wq = (wq_ref[...] * 0.125).astype(jnp.bfloat16)
        wo = wo_ref[...].astype(jnp.bfloat16)
        qi = lax.broadcasted_iota(jnp.int32, (SQ, SKV), 0)
        ki = lax.broadcasted_iota(jnp.int32, (SQ, SKV), 1)
        mask = (jnp.abs(qi - ki) <= 128) | (ki < 32) | (qi < 32)

        started = []

        def send(src, dst, send_sem, recv_sem, tgt):
            rdma = pltpu.make_async_remote_copy(
                src_ref=src, dst_ref=dst, send_sem=send_sem,
                recv_sem=recv_sem, device_id=(tgt,),
                device_id_type=pl.DeviceIdType.MESH,
            )
            rdma.start()
            started.append(rdma)

        def wait_recv(dst, recv_sem):
            pltpu.make_async_remote_copy(
                src_ref=dst, dst_ref=dst, send_sem=recv_sem,
                recv_sem=recv_sem, device_id=(my_pos,),
                device_id_type=pl.DeviceIdType.MESH,
            ).wait_recv()

        for b in range(B):
            xb = x_ref[b].astype(jnp.bfloat16)
            q = jnp.dot(xb, wq, preferred_element_type=jnp.float32)
            ctx_parts = []
            for h in range(HQ_LOCAL):
                qh = q[:, h * DH:(h + 1) * DH].astype(jnp.bfloat16)
                kh = k_ref[b, h]
                s = lax.dot_general(
                    qh, kh, (((1,), (1,)), ((), ())),
                    preferred_element_type=jnp.float32,
                )
                w = jnp.exp(jnp.where(mask, s, -1e9))
                w = w / jnp.sum(w, axis=-1, keepdims=True)
                ctx_parts.append(jnp.dot(
                    w.astype(jnp.bfloat16), v_ref[b, h],
                    preferred_element_type=jnp.float32,
                ))
            ctx = jnp.concatenate(ctx_parts, axis=-1).astype(jnp.bfloat16)
            part = jnp.dot(ctx, wo, preferred_element_type=jnp.float32)
            for t in range(N_DEV):
                send_buf[t, b] = part[t * RQ:(t + 1) * RQ, :].astype(
                    jnp.bfloat16)
            for o in range(1, N_DEV):
                tgt = lax.rem(my_pos + o, N_DEV)
                send(send_buf.at[tgt, b], recv_buf.at[my_pos, b],
                     s1.at[o, b], r1.at[my_pos, b], tgt)

        for b in range(B):
            acc = send_buf[my_pos, b].astype(jnp.float32)
            for o in range(1, N_DEV):
                src = lax.rem(my_pos + N_DEV - o, N_DEV)
                wait_recv(recv_buf.at[src, b], r1.at[src, b])
                acc = acc + recv_buf[src, b].astype(jnp.float32)
            red_buf[b] = acc.astype(jnp.bfloat16)
            out_ref[b, pl.ds(my_pos * RQ, RQ), :] = red_buf[b]
            for o in range(1, N_DEV):
                tgt = lax.rem(my_pos + o, N_DEV)
                send(red_buf.at[b], out_ref.at[b, pl.ds(my_pos * RQ, RQ), :],
                     s2.at[o, b], r2.at[my_pos, b], tgt)

        for b in range(B):
            for o in range(1, N_DEV):
                src = lax.rem(my_pos + N_DEV - o, N_DEV)
                wait_recv(out_ref.at[b, pl.ds(src * RQ, RQ), :], r2.at[src, b])

        for rdma in started:
            rdma.wait_send()

    return pl.pallas_call(
        body,
        out_shape=jax.ShapeDtypeStruct((B, SQ, D_MODEL), jnp.bfloat16),
        in_specs=[pl.BlockSpec(memory_space=pltpu.VMEM)] * 5,
        out_specs=pl.BlockSpec(memory_space=pltpu.VMEM),
        scratch_shapes=[
            pltpu.VMEM((N_DEV, B, RQ, D_MODEL), jnp.bfloat16),
            pltpu.VMEM((N_DEV, B, RQ, D_MODEL), jnp.bfloat16),
            pltpu.VMEM((B, RQ, D_MODEL), jnp.bfloat16),
            pltpu.SemaphoreType.DMA((N_DEV, B)),
            pltpu.SemaphoreType.DMA((N_DEV, B)),
            pltpu.SemaphoreType.DMA((N_DEV, B)),
            pltpu.SemaphoreType.DMA((N_DEV, B)),
        ],
        compiler_params=pltpu.CompilerParams(collective_id=0),
    )(x, Wq, K_loc, V_loc, Wo)


# device time: 17577 ns/iter; 1.3305x vs baseline; 1.2441x over previous
import jax
import jax.numpy as jnp
from jax import lax
from jax.experimental import pallas as pl
from jax.experimental.pallas import tpu as pltpu

N_DEV = 8
B, SQ, SKV = 2, 256, 256
HQ_LOCAL, DH = 4, 64
D_MODEL = 512
RQ = SQ // N_DEV


def kernel(x, Wq, K_ext, V_ext, Wo):
    my = lax.axis_index("i")
    h0 = my * HQ_LOCAL
    K_loc = jnp.transpose(
        lax.dynamic_slice_in_dim(K_ext, h0, HQ_LOCAL, axis=2), (0, 2, 1, 3)
    )
    V_loc = jnp.transpose(
        lax.dynamic_slice_in_dim(V_ext, h0, HQ_LOCAL, axis=2), (0, 2, 1, 3)
    )

    def body(x_ref, wq_ref, k_ref, v_ref, wo_ref, out_ref,
             send_buf, recv_buf, red_buf, s1, r1, s2, r2):
        my_pos = lax.axis_index("i")

        barrier_sem = pltpu.get_barrier_semaphore()
        for o in range(1, N_DEV):
            pl.semaphore_signal(
                barrier_sem, inc=1,
                device_id=(lax.rem(my_pos + o, N_DEV),),
                device_id_type=pl.DeviceIdType.MESH,
            )

        wq = (wq_ref[...] * 0.125).astype(jnp.bfloat16)
        wo = wo_ref[...].astype(jnp.bfloat16)
        qi = lax.broadcasted_iota(jnp.int32, (SQ, SKV), 0)
        ki = lax.broadcasted_iota(jnp.int32, (SQ, SKV), 1)
        mask = (jnp.abs(qi - ki) <= 128) | (ki < 32) | (qi < 32)

        started = []

        def send(src, dst, send_sem, recv_sem, tgt):
            rdma = pltpu.make_async_remote_copy(
                src_ref=src, dst_ref=dst, send_sem=send_sem,
                recv_sem=recv_sem, device_id=(tgt,),
                device_id_type=pl.DeviceIdType.MESH,
            )
            rdma.start()
            started.append(rdma)

        def wait_recv(dst, recv_sem):
            pltpu.make_async_remote_copy(
                src_ref=dst, dst_ref=dst, send_sem=recv_sem,
                recv_sem=recv_sem, device_id=(my_pos,),
                device_id_type=pl.DeviceIdType.MESH,
            ).wait_recv()

        for b in range(B):
            xb = x_ref[b].astype(jnp.bfloat16)
            q = jnp.dot(xb, wq, preferred_element_type=jnp.float32)
            ctx_parts = []
            for h in range(HQ_LOCAL):
                qh = q[:, h * DH:(h + 1) * DH].astype(jnp.bfloat16)
                kh = k_ref[b, h].astype(jnp.bfloat16)
                s = lax.dot_general(
                    qh, kh, (((1,), (1,)), ((), ())),
                    preferred_element_type=jnp.float32,
                )
                w = jnp.exp(jnp.where(mask, s, -1e9))
                w = w / jnp.sum(w, axis=-1, keepdims=True)
                ctx_parts.append(jnp.dot(
                    w.astype(jnp.bfloat16), v_ref[b, h].astype(jnp.bfloat16),
                    preferred_element_type=jnp.float32,
                ))
            ctx = jnp.concatenate(ctx_parts, axis=-1).astype(jnp.bfloat16)
            part = jnp.dot(ctx, wo, preferred_element_type=jnp.float32)
            for t in range(N_DEV):
                send_buf[t, b] = part[t * RQ:(t + 1) * RQ, :].astype(
                    jnp.bfloat16)
            if b == 0:
                pl.semaphore_wait(barrier_sem, N_DEV - 1)
            for o in range(1, N_DEV):
                tgt = lax.rem(my_pos + o, N_DEV)
                send(send_buf.at[tgt, b], recv_buf.at[my_pos, b],
                     s1.at[o, b], r1.at[my_pos, b], tgt)

        for b in range(B):
            acc = send_buf[my_pos, b].astype(jnp.float32)
            for o in range(1, N_DEV):
                src = lax.rem(my_pos + N_DEV - o, N_DEV)
                wait_recv(recv_buf.at[src, b], r1.at[src, b])
                acc = acc + recv_buf[src, b].astype(jnp.float32)
            red_buf[b] = acc.astype(jnp.bfloat16)
            out_ref[b, pl.ds(my_pos * RQ, RQ), :] = red_buf[b]
            for o in range(1, N_DEV):
                tgt = lax.rem(my_pos + o, N_DEV)
                send(red_buf.at[b], out_ref.at[b, pl.ds(my_pos * RQ, RQ), :],
                     s2.at[o, b], r2.at[my_pos, b], tgt)

        for b in range(B):
            for o in range(1, N_DEV):
                src = lax.rem(my_pos + N_DEV - o, N_DEV)
                wait_recv(out_ref.at[b, pl.ds(src * RQ, RQ), :], r2.at[src, b])

        for rdma in started:
            rdma.wait_send()

    return pl.pallas_call(
        body,
        out_shape=jax.ShapeDtypeStruct((B, SQ, D_MODEL), jnp.bfloat16),
        in_specs=[pl.BlockSpec(memory_space=pltpu.VMEM)] * 5,
        out_specs=pl.BlockSpec(memory_space=pltpu.VMEM),
        scratch_shapes=[
            pltpu.VMEM((N_DEV, B, RQ, D_MODEL), jnp.bfloat16),
            pltpu.VMEM((N_DEV, B, RQ, D_MODEL), jnp.bfloat16),
            pltpu.VMEM((B, RQ, D_MODEL), jnp.bfloat16),
            pltpu.SemaphoreType.DMA((N_DEV, B)),
            pltpu.SemaphoreType.DMA((N_DEV, B)),
            pltpu.SemaphoreType.DMA((N_DEV, B)),
            pltpu.SemaphoreType.DMA((N_DEV, B)),
        ],
        compiler_params=pltpu.CompilerParams(collective_id=0),
    )(x, Wq, K_loc, V_loc, Wo)
